# baseline (device time: 46434 ns/iter reference)
import jax
import jax.numpy as jnp
from jax import lax
from jax.experimental import pallas as pl
from jax.experimental.pallas import tpu as pltpu

N_Z = 4


def _allreduce_z(partial):
    t, d = partial.shape

    def body(p_ref, out_ref, comm_ref, send_sems, recv_sems):
        my_x = lax.axis_index("x")
        my_y = lax.axis_index("y")
        my_z = lax.axis_index("z")
        left = (my_z - 1) % N_Z
        right = (my_z + 1) % N_Z

        barrier_sem = pltpu.get_barrier_semaphore()
        for nbr in (left, right):
            pl.semaphore_signal(
                barrier_sem,
                inc=1,
                device_id=(my_x, my_y, nbr),
                device_id_type=pl.DeviceIdType.MESH,
            )
        pl.semaphore_wait(barrier_sem, 2)

        comm_ref[0, :, :] = p_ref[:, :]
        acc = p_ref[:, :]
        for h in range(N_Z - 1):
            rdma = pltpu.make_async_remote_copy(
                src_ref=comm_ref.at[h],
                dst_ref=comm_ref.at[h + 1],
                send_sem=send_sems.at[h],
                recv_sem=recv_sems.at[h + 1],
                device_id=(my_x, my_y, right),
                device_id_type=pl.DeviceIdType.MESH,
            )
            rdma.start()
            rdma.wait()
            acc = acc + comm_ref[h + 1, :, :]
        out_ref[:, :] = acc

    return pl.pallas_call(
        body,
        out_shape=jax.ShapeDtypeStruct((t, d), partial.dtype),
        in_specs=[pl.BlockSpec(memory_space=pltpu.VMEM)],
        out_specs=pl.BlockSpec(memory_space=pltpu.VMEM),
        scratch_shapes=[
            pltpu.VMEM((N_Z, t, d), partial.dtype),
            pltpu.SemaphoreType.DMA((N_Z,)),
            pltpu.SemaphoreType.DMA((N_Z,)),
        ],
        compiler_params=pltpu.CompilerParams(collective_id=0),
    )(partial)


def kernel(ids, E):
    v_per, _ = E.shape
    z = lax.axis_index("z")
    local = ids - z * v_per
    mask = (local >= 0) & (local < v_per)
    safe = jnp.where(mask, local, 0)
    partial = jnp.where(mask[:, None], jnp.take(E, safe, axis=0), 0.0)
    return _allreduce_z(partial.astype(jnp.float32))


# device time: 36486 ns/iter; 1.2727x vs baseline; 1.2727x over previous
import jax
import jax.numpy as jnp
from jax import lax
from jax.experimental import pallas as pl
from jax.experimental.pallas import tpu as pltpu

N_Z = 4
N_DIR = 2


def _allreduce_z(partial):
    t, d = partial.shape
    half = t // 2
    rows = half // N_Z

    def body(p_ref, out_ref, rs_buf, rs_send, rs_recv, ag_send, ag_recv):
        my_x = lax.axis_index("x")
        my_y = lax.axis_index("y")
        my_z = lax.axis_index("z")
        left = (my_z - 1) % N_Z
        right = (my_z + 1) % N_Z
        tgt = (right, left)

        def off(dd, c):
            return dd * half + c * rows

        def c_mod(v):
            return lax.rem(v + 2 * N_Z, N_Z)

        barrier_sem = pltpu.get_barrier_semaphore()
        for nbr in (left, right):
            pl.semaphore_signal(
                barrier_sem,
                inc=1,
                device_id=(my_x, my_y, nbr),
                device_id_type=pl.DeviceIdType.MESH,
            )
        pl.semaphore_wait(barrier_sem, 2)

        out_ref[:, :] = p_ref[:, :]

        for s in range(N_Z - 1):
            rdmas = []
            for dd in range(N_DIR):
                send_c = c_mod(my_z - s) if dd == 0 else c_mod(my_z + s)
                rdma = pltpu.make_async_remote_copy(
                    src_ref=out_ref.at[pl.ds(off(dd, send_c), rows), :],
                    dst_ref=rs_buf.at[dd, s],
                    send_sem=rs_send.at[dd, s],
                    recv_sem=rs_recv.at[dd, s],
                    device_id=(my_x, my_y, tgt[dd]),
                    device_id_type=pl.DeviceIdType.MESH,
                )
                rdma.start()
                rdmas.append(rdma)
            for dd in range(N_DIR):
                recv_c = c_mod(my_z - s - 1) if dd == 0 else c_mod(my_z + s + 1)
                rdmas[dd].wait()
                o = off(dd, recv_c)
                out_ref[pl.ds(o, rows), :] = (
                    out_ref[pl.ds(o, rows), :] + rs_buf[dd, s, :, :]
                )

        for s in range(N_Z - 1):
            sends = []
            recvs = []
            for dd in range(N_DIR):
                send_c = c_mod(my_z + 1 - s) if dd == 0 else c_mod(my_z - 1 + s)
                recv_c = c_mod(my_z - s) if dd == 0 else c_mod(my_z + s)
                so = off(dd, send_c)
                ro = off(dd, recv_c)
                send = pltpu.make_async_remote_copy(
                    src_ref=out_ref.at[pl.ds(so, rows), :],
                    dst_ref=out_ref.at[pl.ds(so, rows), :],
                    send_sem=ag_send.at[dd, s],
                    recv_sem=ag_recv.at[dd, s],
                    device_id=(my_x, my_y, tgt[dd]),
                    device_id_type=pl.DeviceIdType.MESH,
                )
                send.start()
                recv = pltpu.make_async_remote_copy(
                    src_ref=out_ref.at[pl.ds(ro, rows), :],
                    dst_ref=out_ref.at[pl.ds(ro, rows), :],
                    send_sem=ag_send.at[dd, s],
                    recv_sem=ag_recv.at[dd, s],
                    device_id=(my_x, my_y, tgt[dd]),
                    device_id_type=pl.DeviceIdType.MESH,
                )
                sends.append(send)
                recvs.append(recv)
            for dd in range(N_DIR):
                recvs[dd].wait_recv()
            for dd in range(N_DIR):
                sends[dd].wait_send()

    return pl.pallas_call(
        body,
        out_shape=jax.ShapeDtypeStruct((t, d), partial.dtype),
        in_specs=[pl.BlockSpec(memory_space=pltpu.VMEM)],
        out_specs=pl.BlockSpec(memory_space=pltpu.VMEM),
        scratch_shapes=[
            pltpu.VMEM((N_DIR, N_Z - 1, rows, d), partial.dtype),
            pltpu.SemaphoreType.DMA((N_DIR, N_Z - 1)),
            pltpu.SemaphoreType.DMA((N_DIR, N_Z - 1)),
            pltpu.SemaphoreType.DMA((N_DIR, N_Z - 1)),
            pltpu.SemaphoreType.DMA((N_DIR, N_Z - 1)),
        ],
        compiler_params=pltpu.CompilerParams(collective_id=0),
    )(partial)


def kernel(ids, E):
    v_per, _ = E.shape
    z = lax.axis_index("z")
    local = ids - z * v_per
    mask = (local >= 0) & (local < v_per)
    safe = jnp.where(mask, local, 0)
    partial = jnp.where(mask[:, None], jnp.take(E, safe, axis=0), 0.0)
    return _allreduce_z(partial.astype(jnp.float32))
